# baseline (speedup 1.0000x reference)
import functools

import jax
import jax.numpy as jnp
from jax import lax
from jax.experimental import pallas as pl
from jax.experimental.pallas import tpu as pltpu
from jax.experimental.pallas import tpu_sc as plsc

C0 = 0.28209479177387814
INV_C0 = 1.0 / C0
HALF_C0 = 0.5 / C0

NC, NS, L = 2, 16, 16
NW = NC * NS
B, S, H, W = 2, 4, 512, 512
NF = B * S
ROWS = NF * H
RPW = ROWS // NW
G = 8
NGRP = RPW // G
WPF = H // RPW


def _body(img_h, depth_h, par_h, pts_h, hrm_h, dep_h, cnf_h,
          par_v, kx_b, d_b, im_b, p_b, h_b, do_b, cf_b):
    cid = lax.axis_index("c")
    sid = lax.axis_index("s")
    wid = sid * NC + cid
    f = wid // WPF
    row0 = (wid % WPF) * RPW

    pltpu.sync_copy(par_h.at[pl.ds(f * 16 * L, 16 * L)], par_v)

    iota = lax.iota(jnp.int32, L)
    iota_f = iota.astype(jnp.float32)

    def bc(j):
        return par_v[pl.ds(j * L, L)]

    invfx, invfy, cx, cy = bc(0), bc(1), bc(2), bc(3)
    r00, r01, r02 = bc(4), bc(5), bc(6)
    r10, r11, r12 = bc(7), bc(8), bc(9)
    r20, r21, r22 = bc(10), bc(11), bc(12)
    t0, t1, t2 = bc(13), bc(14), bc(15)

    def kx_body(i, carry):
        u = iota_f + (i * L).astype(jnp.float32)
        kx_b[pl.ds(i * L, L)] = (u - cx) * invfx
        return carry

    lax.fori_loop(0, W // L, kx_body, 0)

    lane3 = iota * 3

    def grp_body(g, carry):
        rbase = row0 + g * G
        gbase = (f * H + rbase) * W
        pltpu.sync_copy(depth_h.at[pl.ds(gbase, G * W)], d_b)
        for c in range(3):
            src0 = ((f * 3 + c) * H + rbase) * W
            pltpu.sync_copy(img_h.at[pl.ds(src0, G * W)],
                            im_b.at[pl.ds(c * G * W, G * W)])

        for r in range(G):
            hrow = (rbase + r).astype(jnp.float32)
            kyb = (hrow - cy) * invfy

            def col_body(i, carry, r=r, kyb=kyb):
                base = r * W + i * L
                d = d_b[pl.ds(base, L)]
                kx = kx_b[pl.ds(i * L, L)]
                x = kx * d
                y = kyb * d
                wx = r00 * x + r01 * y + r02 * d + t0
                wy = r10 * x + r11 * y + r12 * d + t1
                wz = r20 * x + r21 * y + r22 * d + t2
                mf = (d > 0.0).astype(jnp.float32)
                colb = r * (3 * W) + i * (3 * L) + lane3
                plsc.store_scatter(p_b, [colb], wx * mf)
                plsc.store_scatter(p_b, [colb + 1], wy * mf)
                plsc.store_scatter(p_b, [colb + 2], wz * mf)
                sc = mf * INV_C0
                for c in range(3):
                    ic = im_b[pl.ds(c * G * W + base, L)]
                    plsc.store_scatter(h_b, [colb + c], ic * sc - HALF_C0)
                do_b[pl.ds(base, L)] = d
                cf_b[pl.ds(base, L)] = mf
                return carry

            lax.fori_loop(0, W // L, col_body, 0)

        pltpu.sync_copy(p_b, pts_h.at[pl.ds(gbase * 3, G * W * 3)])
        pltpu.sync_copy(h_b, hrm_h.at[pl.ds(gbase * 3, G * W * 3)])
        pltpu.sync_copy(do_b, dep_h.at[pl.ds(gbase, G * W)])
        pltpu.sync_copy(cf_b, cnf_h.at[pl.ds(gbase, G * W)])
        return carry

    lax.fori_loop(0, NGRP, grp_body, 0)


def kernel(img, timestamp, is_static, depth, c2w, intrinsics):
    del timestamp, is_static
    imgf = img.astype(jnp.float32).reshape(-1)
    depthf = depth.astype(jnp.float32).reshape(-1)
    Kf = intrinsics.astype(jnp.float32).reshape(NF, 3, 3)
    c2wf = c2w.astype(jnp.float32).reshape(NF, 4, 4)
    params = jnp.concatenate([
        1.0 / Kf[:, 0, 0:1], 1.0 / Kf[:, 1, 1:2],
        Kf[:, 0, 2:3], Kf[:, 1, 2:3],
        c2wf[:, :3, :3].reshape(NF, 9),
        c2wf[:, :3, 3],
    ], axis=1)
    params = jnp.broadcast_to(params[:, :, None],
                              (NF, 16, L)).reshape(-1)

    npx = NF * H * W
    mesh = plsc.VectorSubcoreMesh(core_axis_name="c", subcore_axis_name="s",
                                  num_cores=NC, num_subcores=NS)
    run = pl.kernel(
        _body,
        out_type=[
            jax.ShapeDtypeStruct((npx * 3,), jnp.float32),
            jax.ShapeDtypeStruct((npx * 3,), jnp.float32),
            jax.ShapeDtypeStruct((npx,), jnp.float32),
            jax.ShapeDtypeStruct((npx,), jnp.float32),
        ],
        mesh=mesh,
        compiler_params=pltpu.CompilerParams(needs_layout_passes=False),
        scratch_types=[
            pltpu.VMEM((16 * L,), jnp.float32),
            pltpu.VMEM((W,), jnp.float32),
            pltpu.VMEM((G * W,), jnp.float32),
            pltpu.VMEM((3 * G * W,), jnp.float32),
            pltpu.VMEM((G * W * 3,), jnp.float32),
            pltpu.VMEM((G * W * 3,), jnp.float32),
            pltpu.VMEM((G * W,), jnp.float32),
            pltpu.VMEM((G * W,), jnp.float32),
        ],
    )
    pts, hrm, dep, cnf = run(imgf, depthf, params)
    return (pts.reshape(B, S, H, W, 3),
            hrm.reshape(B, S, H, W, 1, 3),
            dep.reshape(B, S, H, W, 1),
            cnf.reshape(B, S, H, W, 1))


# speedup vs baseline: 12.7834x; 12.7834x over previous
import jax
import jax.numpy as jnp
from jax import lax
from jax.experimental import pallas as pl
from jax.experimental.pallas import tpu as pltpu
from jax.experimental.pallas import tpu_sc as plsc

C0 = 0.28209479177387814
INV_C0 = 1.0 / C0
HALF_C0 = 0.5 / C0

NC, NS, L = 2, 16, 16
NW = NC * NS
B, S, H, W = 2, 4, 512, 512
NF = B * S
G = 8
NT = H // G
TPW = NF * NT // NW
WPF = NT // TPW
GSZ = G * W


def _body(img_h, depth_h, par_h, pts_h, hrm_h, dep_h, cnf_h,
          par_v, kx_b, d_b, im_b, p_b, h_b, do_b, cf_b):
    cid = lax.axis_index("c")
    sid = lax.axis_index("s")
    wid = sid * NC + cid
    f = wid // WPF
    t0_ = (wid % WPF) * TPW

    pltpu.sync_copy(par_h.at[pl.ds(f * 16 * L, 16 * L)], par_v)

    iota = lax.iota(jnp.int32, L)
    iota_f = iota.astype(jnp.float32)

    def bc(j):
        return par_v[pl.ds(j * L, L)]

    invfx, invfy, cx, cy = bc(0), bc(1), bc(2), bc(3)
    r00, r01, r02 = bc(4), bc(5), bc(6)
    r10, r11, r12 = bc(7), bc(8), bc(9)
    r20, r21, r22 = bc(10), bc(11), bc(12)
    t0, t1, t2 = bc(13), bc(14), bc(15)

    def kx_body(i, carry):
        u = iota_f + (i * L).astype(jnp.float32)
        kx_b[pl.ds(i * L, L)] = (u - cx) * invfx
        return carry

    lax.fori_loop(0, W // L, kx_body, 0)

    def grp_body(g, carry):
        h8 = t0_ + g
        base = (f * NT + h8) * GSZ
        pltpu.sync_copy(depth_h.at[pl.ds(base, GSZ)], d_b)
        for c in range(3):
            pltpu.sync_copy(img_h.at[pl.ds(((f * 3 + c) * NT + h8) * GSZ, GSZ)],
                            im_b.at[pl.ds(c * GSZ, GSZ)])

        for r in range(G):
            hrow = (h8 * G + r).astype(jnp.float32)
            kyb = (hrow - cy) * invfy

            def col_body(i, carry, r=r, kyb=kyb):
                wt = i >> 3
                io = i & 7
                off = wt * 1024 + r * 128 + io * L
                wb = i * L
                d = d_b[pl.ds(off, L)]
                kx = kx_b[pl.ds(wb, L)]
                x = kx * d
                y = kyb * d
                wx = r00 * x + r01 * y + r02 * d + t0
                wy = r10 * x + r11 * y + r12 * d + t1
                wz = r20 * x + r21 * y + r22 * d + t2
                mf = (d > 0.0).astype(jnp.float32)
                p_b[pl.ds(off, L)] = wx * mf
                p_b[pl.ds(GSZ + off, L)] = wy * mf
                p_b[pl.ds(2 * GSZ + off, L)] = wz * mf
                sc = mf * INV_C0
                for c in range(3):
                    ic = im_b[pl.ds(c * GSZ + off, L)]
                    h_b[pl.ds(r * 1536 + c * W + wb, L)] = ic * sc - HALF_C0
                do_b[pl.ds(r * W + wb, L)] = d
                cf_b[pl.ds(r * W + wb, L)] = mf
                return carry

            lax.fori_loop(0, W // L, col_body, 0)

        for c in range(3):
            pltpu.sync_copy(p_b.at[pl.ds(c * GSZ, GSZ)],
                            pts_h.at[pl.ds(((f * 3 + c) * NT + h8) * GSZ, GSZ)])
        pltpu.sync_copy(h_b, hrm_h.at[pl.ds(base * 3, 3 * GSZ)])
        pltpu.sync_copy(do_b, dep_h.at[pl.ds(base, GSZ)])
        pltpu.sync_copy(cf_b, cnf_h.at[pl.ds(base, GSZ)])
        return carry

    lax.fori_loop(0, TPW, grp_body, 0)


def kernel(img, timestamp, is_static, depth, c2w, intrinsics):
    del timestamp, is_static
    imgf = (img.astype(jnp.float32)
            .reshape(B, S, 3, NT, G, W // 128, 128)
            .transpose(0, 1, 2, 3, 5, 4, 6)
            .reshape(-1))
    depthf = (depth.astype(jnp.float32)
              .reshape(B, S, NT, G, W // 128, 128)
              .transpose(0, 1, 2, 4, 3, 5)
              .reshape(-1))
    Kf = intrinsics.astype(jnp.float32).reshape(NF, 3, 3)
    c2wf = c2w.astype(jnp.float32).reshape(NF, 4, 4)
    params = jnp.concatenate([
        1.0 / Kf[:, 0, 0:1], 1.0 / Kf[:, 1, 1:2],
        Kf[:, 0, 2:3], Kf[:, 1, 2:3],
        c2wf[:, :3, :3].reshape(NF, 9),
        c2wf[:, :3, 3],
    ], axis=1)
    params = jnp.broadcast_to(params[:, :, None],
                              (NF, 16, L)).reshape(-1)

    npx = NF * H * W
    mesh = plsc.VectorSubcoreMesh(core_axis_name="c", subcore_axis_name="s",
                                  num_cores=NC, num_subcores=NS)
    run = pl.kernel(
        _body,
        out_type=[
            jax.ShapeDtypeStruct((npx * 3,), jnp.float32),
            jax.ShapeDtypeStruct((npx * 3,), jnp.float32),
            jax.ShapeDtypeStruct((npx,), jnp.float32),
            jax.ShapeDtypeStruct((npx,), jnp.float32),
        ],
        mesh=mesh,
        compiler_params=pltpu.CompilerParams(needs_layout_passes=False),
        scratch_types=[
            pltpu.VMEM((16 * L,), jnp.float32),
            pltpu.VMEM((W,), jnp.float32),
            pltpu.VMEM((GSZ,), jnp.float32),
            pltpu.VMEM((3 * GSZ,), jnp.float32),
            pltpu.VMEM((3 * GSZ,), jnp.float32),
            pltpu.VMEM((3 * GSZ,), jnp.float32),
            pltpu.VMEM((GSZ,), jnp.float32),
            pltpu.VMEM((GSZ,), jnp.float32),
        ],
    )
    pts, hrm, dep, cnf = run(imgf, depthf, params)
    pts = (pts.reshape(B, S, 3, NT, W // 128, G, 128)
           .transpose(0, 1, 3, 5, 4, 6, 2)
           .reshape(B, S, H, W, 3))
    hrm = (hrm.reshape(B, S, H, 3, 1, W)
           .transpose(0, 1, 2, 5, 4, 3))
    return (pts, hrm,
            dep.reshape(B, S, H, W, 1),
            cnf.reshape(B, S, H, W, 1))


# speedup vs baseline: 19.3669x; 1.5150x over previous
import jax
import jax.numpy as jnp
from jax import lax
from jax.experimental import pallas as pl
from jax.experimental.pallas import tpu as pltpu
from jax.experimental.pallas import tpu_sc as plsc

C0 = 0.28209479177387814
INV_C0 = 1.0 / C0
HALF_C0 = 0.5 / C0

NC, NS, L = 2, 16, 16
NW = NC * NS
B, S, H, W = 2, 4, 512, 512
NF = B * S
G = 8
NT = H // G
TPW = NF * NT // NW
WPF = NT // TPW
GSZ = G * W


def _body(img_h, depth_h, par_h, pts_h, hrm_h, dep_h, cnf_h,
          par_v, kx_b,
          d_b0, d_b1, im_b0, im_b1, p_b0, p_b1, h_b0, h_b1,
          do_b0, do_b1, cf_b0, cf_b1,
          in_s0, in_s1, out_s0, out_s1):
    d_b = (d_b0, d_b1)
    im_b = (im_b0, im_b1)
    p_b = (p_b0, p_b1)
    h_b = (h_b0, h_b1)
    do_b = (do_b0, do_b1)
    cf_b = (cf_b0, cf_b1)
    in_s = (in_s0, in_s1)
    out_s = (out_s0, out_s1)

    cid = lax.axis_index("c")
    sid = lax.axis_index("s")
    wid = sid * NC + cid
    f = wid // WPF
    t0_ = (wid % WPF) * TPW

    pltpu.sync_copy(par_h.at[pl.ds(f * 16 * L, 16 * L)], par_v)

    iota = lax.iota(jnp.int32, L)
    iota_f = iota.astype(jnp.float32)

    def bc(j):
        return par_v[pl.ds(j * L, L)]

    invfx, invfy, cx, cy = bc(0), bc(1), bc(2), bc(3)
    r00, r01, r02 = bc(4), bc(5), bc(6)
    r10, r11, r12 = bc(7), bc(8), bc(9)
    r20, r21, r22 = bc(10), bc(11), bc(12)
    t0, t1, t2 = bc(13), bc(14), bc(15)

    def kx_body(i, carry):
        u = iota_f + (i * L).astype(jnp.float32)
        kx_b[pl.ds(i * L, L)] = (u - cx) * invfx
        return carry

    lax.fori_loop(0, W // L, kx_body, 0)

    def issue_in(g, slot):
        h8 = t0_ + g
        pltpu.async_copy(depth_h.at[pl.ds((f * NT + h8) * GSZ, GSZ)],
                         d_b[slot], in_s[slot])
        for c in range(3):
            pltpu.async_copy(img_h.at[pl.ds(((f * 3 + c) * NT + h8) * GSZ, GSZ)],
                             im_b[slot].at[pl.ds(c * GSZ, GSZ)], in_s[slot])

    def wait_in(slot):
        pltpu.make_async_copy(depth_h.at[pl.ds(0, GSZ)], d_b[slot],
                              in_s[slot]).wait()
        pltpu.make_async_copy(img_h.at[pl.ds(0, 3 * GSZ)], im_b[slot],
                              in_s[slot]).wait()

    def issue_out(g, slot):
        h8 = t0_ + g
        base = (f * NT + h8) * GSZ
        for c in range(3):
            pltpu.async_copy(p_b[slot].at[pl.ds(c * GSZ, GSZ)],
                             pts_h.at[pl.ds(((f * 3 + c) * NT + h8) * GSZ, GSZ)],
                             out_s[slot])
        pltpu.async_copy(h_b[slot], hrm_h.at[pl.ds(base * 3, 3 * GSZ)], out_s[slot])
        pltpu.async_copy(do_b[slot], dep_h.at[pl.ds(base, GSZ)], out_s[slot])
        pltpu.async_copy(cf_b[slot], cnf_h.at[pl.ds(base, GSZ)], out_s[slot])

    def wait_out(slot):
        pltpu.make_async_copy(p_b[slot], pts_h.at[pl.ds(0, 3 * GSZ)],
                              out_s[slot]).wait()
        pltpu.make_async_copy(h_b[slot], hrm_h.at[pl.ds(0, 3 * GSZ)],
                              out_s[slot]).wait()
        pltpu.make_async_copy(do_b[slot], dep_h.at[pl.ds(0, GSZ)],
                              out_s[slot]).wait()
        pltpu.make_async_copy(cf_b[slot], cnf_h.at[pl.ds(0, GSZ)],
                              out_s[slot]).wait()

    def compute(g, slot):
        h8 = t0_ + g
        for r in range(G):
            hrow = (h8 * G + r).astype(jnp.float32)
            kyb = (hrow - cy) * invfy

            def col_body(i, carry, r=r, kyb=kyb):
                wt = i >> 3
                io = i & 7
                off = wt * 1024 + r * 128 + io * L
                wb = i * L
                d = d_b[slot][pl.ds(off, L)]
                kx = kx_b[pl.ds(wb, L)]
                x = kx * d
                y = kyb * d
                wx = r00 * x + r01 * y + r02 * d + t0
                wy = r10 * x + r11 * y + r12 * d + t1
                wz = r20 * x + r21 * y + r22 * d + t2
                mf = (d > 0.0).astype(jnp.float32)
                p_b[slot][pl.ds(off, L)] = wx * mf
                p_b[slot][pl.ds(GSZ + off, L)] = wy * mf
                p_b[slot][pl.ds(2 * GSZ + off, L)] = wz * mf
                sc = mf * INV_C0
                for c in range(3):
                    ic = im_b[slot][pl.ds(c * GSZ + off, L)]
                    h_b[slot][pl.ds(r * 1536 + c * W + wb, L)] = ic * sc - HALF_C0
                do_b[slot][pl.ds(r * W + wb, L)] = d
                cf_b[slot][pl.ds(r * W + wb, L)] = mf
                return carry

            lax.fori_loop(0, W // L, col_body, 0)

    issue_in(0, 0)
    issue_in(1, 1)

    def grp_body(g2, carry):
        for slot in (0, 1):
            g = g2 * 2 + slot
            wait_in(slot)

            @pl.when(g2 >= 1)
            def _():
                wait_out(slot)

            compute(g, slot)

            @pl.when(g < TPW - 2)
            def _():
                issue_in(g + 2, slot)

            issue_out(g, slot)
        return carry

    lax.fori_loop(0, TPW // 2, grp_body, 0)
    wait_out(0)
    wait_out(1)


def kernel(img, timestamp, is_static, depth, c2w, intrinsics):
    del timestamp, is_static
    imgf = (img.astype(jnp.float32)
            .reshape(B, S, 3, NT, G, W // 128, 128)
            .transpose(0, 1, 2, 3, 5, 4, 6)
            .reshape(-1))
    depthf = (depth.astype(jnp.float32)
              .reshape(B, S, NT, G, W // 128, 128)
              .transpose(0, 1, 2, 4, 3, 5)
              .reshape(-1))
    Kf = intrinsics.astype(jnp.float32).reshape(NF, 3, 3)
    c2wf = c2w.astype(jnp.float32).reshape(NF, 4, 4)
    params = jnp.concatenate([
        1.0 / Kf[:, 0, 0:1], 1.0 / Kf[:, 1, 1:2],
        Kf[:, 0, 2:3], Kf[:, 1, 2:3],
        c2wf[:, :3, :3].reshape(NF, 9),
        c2wf[:, :3, 3],
    ], axis=1)
    params = jnp.broadcast_to(params[:, :, None],
                              (NF, 16, L)).reshape(-1)

    npx = NF * H * W
    mesh = plsc.VectorSubcoreMesh(core_axis_name="c", subcore_axis_name="s",
                                  num_cores=NC, num_subcores=NS)
    run = pl.kernel(
        _body,
        out_type=[
            jax.ShapeDtypeStruct((npx * 3,), jnp.float32),
            jax.ShapeDtypeStruct((npx * 3,), jnp.float32),
            jax.ShapeDtypeStruct((npx,), jnp.float32),
            jax.ShapeDtypeStruct((npx,), jnp.float32),
        ],
        mesh=mesh,
        compiler_params=pltpu.CompilerParams(needs_layout_passes=False),
        scratch_types=(
            [pltpu.VMEM((16 * L,), jnp.float32),
             pltpu.VMEM((W,), jnp.float32)]
            + [pltpu.VMEM((GSZ,), jnp.float32)] * 2
            + [pltpu.VMEM((3 * GSZ,), jnp.float32)] * 2
            + [pltpu.VMEM((3 * GSZ,), jnp.float32)] * 2
            + [pltpu.VMEM((3 * GSZ,), jnp.float32)] * 2
            + [pltpu.VMEM((GSZ,), jnp.float32)] * 2
            + [pltpu.VMEM((GSZ,), jnp.float32)] * 2
            + [pltpu.SemaphoreType.DMA] * 4
        ),
    )
    pts, hrm, dep, cnf = run(imgf, depthf, params)
    pts = (pts.reshape(B, S, 3, NT, W // 128, G, 128)
           .transpose(0, 1, 3, 5, 4, 6, 2)
           .reshape(B, S, H, W, 3))
    hrm = (hrm.reshape(B, S, H, 3, 1, W)
           .transpose(0, 1, 2, 5, 4, 3))
    return (pts, hrm,
            dep.reshape(B, S, H, W, 1),
            cnf.reshape(B, S, H, W, 1))
